# baseline (speedup 1.0000x reference)
import jax
import jax.numpy as jnp
import numpy as np
from jax import lax
from jax.experimental import pallas as pl
from jax.experimental.pallas import tpu as pltpu

C = 256
K = 51


def _flip_rows(t):
    u = jnp.concatenate(
        [t[C - 8 * (g + 1):C - 8 * g] for g in range(C // 8)], axis=0
    )
    r = lax.broadcasted_iota(jnp.int32, (C, K), 0)
    for s in (4, 2, 1):
        down = pltpu.roll(u, s, 0)
        up = pltpu.roll(u, C - s, 0)
        u = jnp.where((r & s) != 0, down, up)
    return u


def _body(x1_ref, x2_ref, jk_ref, o_ref):
    m = pl.program_id(1)
    a = jnp.where(m == 0, x1_ref[0], x2_ref[0])
    b = jnp.where(m == 0, x2_ref[0], x1_ref[0])
    t = jnp.dot(b, jk_ref[...], preferred_element_type=jnp.float32)
    o_ref[...] = ((a + _flip_rows(t)) * 0.5)[None]


def kernel(x, in_inv, out_inv):
    del in_inv, out_inv
    jk = jnp.asarray(np.fliplr(np.eye(K, dtype=np.float32)))
    return pl.pallas_call(
        _body,
        grid=(C // 2, 2),
        in_specs=[
            pl.BlockSpec((1, C, K), lambda h, m: (h, 0, 0)),
            pl.BlockSpec((1, C, K), lambda h, m: (C - 1 - h, 0, 0)),
            pl.BlockSpec((K, K), lambda h, m: (0, 0)),
        ],
        out_specs=pl.BlockSpec(
            (1, C, K), lambda h, m: (h * (1 - m) + (C - 1 - h) * m, 0, 0)
        ),
        out_shape=jax.ShapeDtypeStruct((C, C, K), jnp.float32),
    )(x, x, jk)


# speedup vs baseline: 1.8324x; 1.8324x over previous
import jax
import jax.numpy as jnp
import numpy as np
from jax import lax
from jax.experimental import pallas as pl
from jax.experimental.pallas import tpu as pltpu

C = 256
K = 51
B = 4
R = B * C
NB = C // B


def _flip_rows(t):
    u = jnp.concatenate(
        [t[R - 8 * (g + 1):R - 8 * g] for g in range(R // 8)], axis=0
    )
    r = lax.broadcasted_iota(jnp.int32, (R, K), 0)
    for s in (4, 2, 1):
        down = pltpu.roll(u, s, 0)
        up = pltpu.roll(u, R - s, 0)
        u = jnp.where((r & s) != 0, down, up)
    return u


def _body(x1_ref, x2_ref, jk_ref, o_ref):
    m = pl.program_id(1)
    a = jnp.where(m == 0, x1_ref[...], x2_ref[...]).reshape(R, K)
    b = jnp.where(m == 0, x2_ref[...], x1_ref[...]).reshape(R, K)
    t = jnp.dot(b, jk_ref[...], preferred_element_type=jnp.float32)
    o_ref[...] = ((a + _flip_rows(t)) * 0.5).reshape(B, C, K)


def kernel(x, in_inv, out_inv):
    del in_inv, out_inv
    jk = jnp.asarray(np.fliplr(np.eye(K, dtype=np.float32)))
    return pl.pallas_call(
        _body,
        grid=(NB // 2, 2),
        in_specs=[
            pl.BlockSpec((B, C, K), lambda h, m: (h, 0, 0)),
            pl.BlockSpec((B, C, K), lambda h, m: (NB - 1 - h, 0, 0)),
            pl.BlockSpec((K, K), lambda h, m: (0, 0)),
        ],
        out_specs=pl.BlockSpec(
            (B, C, K), lambda h, m: (h * (1 - m) + (NB - 1 - h) * m, 0, 0)
        ),
        out_shape=jax.ShapeDtypeStruct((C, C, K), jnp.float32),
    )(x, x, jk)


# speedup vs baseline: 1.9898x; 1.0859x over previous
import jax
import jax.numpy as jnp
import numpy as np
from jax import lax
from jax.experimental import pallas as pl
from jax.experimental.pallas import tpu as pltpu

C = 256
K = 51
B = 8
R = B * C
NB = C // B
_S = (4, 2, 1)


def _flip_rows(t, masks):
    u = jnp.concatenate(
        [t[R - 8 * (g + 1):R - 8 * g] for g in range(R // 8)], axis=0
    )
    for s, mask in zip(_S, masks):
        down = pltpu.roll(u, s, 0)
        up = pltpu.roll(u, R - s, 0)
        u = jnp.where(mask, down, up)
    return u


def _body(x1_ref, x2_ref, jk_ref, m0_ref, m1_ref, m2_ref, o_ref):
    m = pl.program_id(1)
    a = jnp.where(m == 0, x1_ref[...], x2_ref[...]).reshape(R, K)
    b = jnp.where(m == 0, x2_ref[...], x1_ref[...]).reshape(R, K)
    t = jnp.dot(b, jk_ref[...], preferred_element_type=jnp.float32)
    masks = (m0_ref[...], m1_ref[...], m2_ref[...])
    o_ref[...] = ((a + _flip_rows(t, masks)) * 0.5).reshape(B, C, K)


def kernel(x, in_inv, out_inv):
    del in_inv, out_inv
    jk = jnp.asarray(np.fliplr(np.eye(K, dtype=np.float32)))
    rows = np.arange(R)[:, None]
    masks = [jnp.asarray(np.broadcast_to((rows & s) != 0, (R, K))) for s in _S]
    const_spec = lambda shape: pl.BlockSpec(shape, lambda h, m: (0,) * len(shape))
    return pl.pallas_call(
        _body,
        grid=(NB // 2, 2),
        in_specs=[
            pl.BlockSpec((B, C, K), lambda h, m: (h, 0, 0)),
            pl.BlockSpec((B, C, K), lambda h, m: (NB - 1 - h, 0, 0)),
            const_spec((K, K)),
            const_spec((R, K)),
            const_spec((R, K)),
            const_spec((R, K)),
        ],
        out_specs=pl.BlockSpec(
            (B, C, K), lambda h, m: (h * (1 - m) + (NB - 1 - h) * m, 0, 0)
        ),
        out_shape=jax.ShapeDtypeStruct((C, C, K), jnp.float32),
    )(x, x, jk, *masks)
